# Initial kernel scaffold; baseline (speedup 1.0000x reference)
import jax
import jax.numpy as jnp
from jax import lax
from jax.experimental import pallas as pl
from jax.experimental.pallas import tpu as pltpu
from jax.experimental.pallas import tpu_sc as plsc

B = 16384
L = 20
D = 64
DC = 128
TOK = 64

NC = 2
NS = 16
NW = NC * NS
PB = B // NW
CB = 32
NCH = PB // CB
RPC = CB * L
GID = 128
NG = RPC // GID


def _sc_body(idx_hbm, mask_hbm, llmidx_hbm, tool_tab, llm_tab,
             tm_out, le_out,
             idx_v, mask_v, rows_v, tm_v, lidx_v, lrows_v, sem):
    c = lax.axis_index("c")
    s = lax.axis_index("s")
    wid = s * NC + c

    def chunk(ci, carry):
        base = wid * PB + ci * CB
        row0 = (wid * NCH + ci) * NG
        pltpu.sync_copy(idx_hbm.at[pl.ds(row0, NG)], idx_v)
        pltpu.sync_copy(mask_hbm.at[pl.ds(base * L, RPC)], mask_v)
        pltpu.sync_copy(llmidx_hbm.at[pl.ds(base, CB)], lidx_v)
        descs = [
            pltpu.async_copy(tool_tab.at[idx_v.at[j]],
                             rows_v.at[pl.ds(j * GID, GID)], sem)
            for j in range(NG)
        ]
        ldesc = pltpu.async_copy(llm_tab.at[lidx_v], lrows_v, sem)
        for dsc in descs:
            dsc.wait()
        ldesc.wait()

        def bbody(b, bc):
            r0 = b * L
            denom = jnp.float32(1e-8)
            accs = [jnp.zeros((16,), jnp.float32) for _ in range(4)]
            for l in range(L):
                w = mask_v[r0 + l]
                denom = denom + w
                for g in range(4):
                    accs[g] = accs[g] + w * rows_v[r0 + l, pl.ds(g * 16, 16)]
            inv = 1.0 / denom
            for g in range(4):
                tm_v[b, pl.ds(g * 16, 16)] = accs[g] * inv
            return bc

        lax.fori_loop(0, CB, bbody, 0)
        pltpu.sync_copy(tm_v, tm_out.at[pl.ds(base, CB)])
        pltpu.sync_copy(lrows_v, le_out.at[pl.ds(base, CB)])
        return carry

    lax.fori_loop(0, NCH, chunk, 0)


_sc_pool = pl.kernel(
    _sc_body,
    out_type=(
        jax.ShapeDtypeStruct((B, D), jnp.float32),
        jax.ShapeDtypeStruct((B, D), jnp.float32),
    ),
    mesh=plsc.VectorSubcoreMesh(core_axis_name="c", subcore_axis_name="s"),
    scratch_types=[
        pltpu.VMEM((NG, GID), jnp.int32),
        pltpu.VMEM((RPC,), jnp.float32),
        pltpu.VMEM((RPC, D), jnp.float32),
        pltpu.VMEM((CB, D), jnp.float32),
        pltpu.VMEM((CB,), jnp.int32),
        pltpu.VMEM((CB, D), jnp.float32),
        pltpu.SemaphoreType.DMA,
    ],
)


def _tc_body(a_ref, le_ref, tm_ref, wc_ref, wi_ref, o_ref):
    e = jnp.dot(a_ref[...], wc_ref[...], preferred_element_type=jnp.float32)
    ids = jnp.concatenate([le_ref[...], tm_ref[...]], axis=-1)
    e = e + jnp.dot(ids, wi_ref[...], preferred_element_type=jnp.float32)
    n = jnp.sqrt(jnp.sum(e * e, axis=-1, keepdims=True))
    o_ref[...] = e / jnp.maximum(n, 1e-12)


_BLK = 2048


def _tc_dense(A_content, le, tm, wc_t, wi_t):
    grid = (B // _BLK,)
    return pl.pallas_call(
        _tc_body,
        grid=grid,
        in_specs=[
            pl.BlockSpec((_BLK, DC), lambda i: (i, 0)),
            pl.BlockSpec((_BLK, D), lambda i: (i, 0)),
            pl.BlockSpec((_BLK, D), lambda i: (i, 0)),
            pl.BlockSpec((DC, TOK), lambda i: (0, 0)),
            pl.BlockSpec((2 * D, TOK), lambda i: (0, 0)),
        ],
        out_specs=pl.BlockSpec((_BLK, TOK), lambda i: (i, 0)),
        out_shape=jax.ShapeDtypeStruct((B, TOK), jnp.float32),
    )(A_content, le, tm, wc_t, wi_t)


def kernel(A_content, tool_idx_pad, tool_mask, llm_idx, emb_tool, emb_llm,
           W_content, W_ids):
    idx2d = tool_idx_pad.astype(jnp.int32).reshape(B * L // GID, GID)
    maskf = tool_mask.reshape(B * L)
    llmi = llm_idx.astype(jnp.int32)
    tm, le = _sc_pool(idx2d, maskf, llmi, emb_tool, emb_llm)
    return _tc_dense(A_content, le, tm, W_content.T, W_ids.T)


# speedup vs baseline: 6.6777x; 6.6777x over previous
import jax
import jax.numpy as jnp
from jax import lax
from jax.experimental import pallas as pl
from jax.experimental.pallas import tpu as pltpu
from jax.experimental.pallas import tpu_sc as plsc

B = 16384
L = 20
D = 64
DC = 128
TOK = 64

NC = 2
NS = 16
NW = NC * NS
PB = B // NW
CB = 32
NCH = PB // CB
RPC = CB * L
GID = 128
NG = RPC // GID


def _sc_body(idx_hbm, mask_hbm, llmidx_hbm, tool_tab, llm_tab,
             tm_out, le_out,
             idx_v, mask_v, rows_v, tm_v, lidx_v, lrows_v, sem):
    c = lax.axis_index("c")
    s = lax.axis_index("s")
    wid = s * NC + c

    def chunk(ci, carry):
        base = wid * PB + ci * CB
        pltpu.sync_copy(idx_hbm.at[pl.ds(base * L, RPC)], idx_v)
        pltpu.sync_copy(mask_hbm.at[pl.ds(base * L, RPC)],
                        mask_v.at[pl.ds(0, RPC)])
        pltpu.sync_copy(llmidx_hbm.at[pl.ds(base, CB)], lidx_v)
        descs = [
            pltpu.async_copy(tool_tab.at[idx_v.at[pl.ds(j * GID, GID)]],
                             rows_v.at[pl.ds(j * GID, GID)], sem)
            for j in range(NG)
        ]
        ldesc = pltpu.async_copy(llm_tab.at[lidx_v], lrows_v, sem)
        for dsc in descs:
            dsc.wait()
        ldesc.wait()

        def bbody(b, bc):
            r0 = b * L
            m0 = mask_v[pl.ds(r0, 16)]
            m1 = mask_v[pl.ds(r0 + 16, 16)]
            ws = [m0[l] for l in range(16)] + [m1[l] for l in range(L - 16)]
            denom = jnp.float32(1e-8)
            for l in range(L):
                denom = denom + ws[l]
            inv = jnp.full((16,), 1.0, jnp.float32) / jnp.broadcast_to(denom, (16,))
            accs = [jnp.zeros((16,), jnp.float32) for _ in range(4)]
            for l in range(L):
                for g in range(4):
                    accs[g] = accs[g] + ws[l] * rows_v[r0 + l, pl.ds(g * 16, 16)]
            for g in range(4):
                tm_v[b, pl.ds(g * 16, 16)] = accs[g] * inv
            return bc

        lax.fori_loop(0, CB, bbody, 0)
        pltpu.sync_copy(tm_v, tm_out.at[pl.ds(base, CB)])
        pltpu.sync_copy(lrows_v, le_out.at[pl.ds(base, CB)])
        return carry

    lax.fori_loop(0, NCH, chunk, 0)


_sc_pool = pl.kernel(
    _sc_body,
    out_type=(
        jax.ShapeDtypeStruct((B, D), jnp.float32),
        jax.ShapeDtypeStruct((B, D), jnp.float32),
    ),
    mesh=plsc.VectorSubcoreMesh(core_axis_name="c", subcore_axis_name="s"),
    compiler_params=pltpu.CompilerParams(use_tc_tiling_on_sc=False),
    scratch_types=[
        pltpu.VMEM((RPC,), jnp.int32),
        pltpu.VMEM((RPC + 16,), jnp.float32),
        pltpu.VMEM((RPC, D), jnp.float32),
        pltpu.VMEM((CB, D), jnp.float32),
        pltpu.VMEM((CB,), jnp.int32),
        pltpu.VMEM((CB, D), jnp.float32),
        pltpu.SemaphoreType.DMA,
    ],
)


def _tc_body(a_ref, le_ref, tm_ref, wc_ref, wi_ref, o_ref):
    e = jnp.dot(a_ref[...], wc_ref[...], preferred_element_type=jnp.float32)
    ids = jnp.concatenate([le_ref[...], tm_ref[...]], axis=-1)
    e = e + jnp.dot(ids, wi_ref[...], preferred_element_type=jnp.float32)
    n = jnp.sqrt(jnp.sum(e * e, axis=-1, keepdims=True))
    o_ref[...] = e / jnp.maximum(n, 1e-12)


_BLK = 2048


def _tc_dense(A_content, le, tm, wc_t, wi_t):
    grid = (B // _BLK,)
    return pl.pallas_call(
        _tc_body,
        grid=grid,
        in_specs=[
            pl.BlockSpec((_BLK, DC), lambda i: (i, 0)),
            pl.BlockSpec((_BLK, D), lambda i: (i, 0)),
            pl.BlockSpec((_BLK, D), lambda i: (i, 0)),
            pl.BlockSpec((DC, TOK), lambda i: (0, 0)),
            pl.BlockSpec((2 * D, TOK), lambda i: (0, 0)),
        ],
        out_specs=pl.BlockSpec((_BLK, TOK), lambda i: (i, 0)),
        out_shape=jax.ShapeDtypeStruct((B, TOK), jnp.float32),
    )(A_content, le, tm, wc_t, wi_t)


def kernel(A_content, tool_idx_pad, tool_mask, llm_idx, emb_tool, emb_llm,
           W_content, W_ids):
    idx2d = tool_idx_pad.astype(jnp.int32).reshape(B * L)
    maskf = tool_mask.reshape(B * L)
    llmi = llm_idx.astype(jnp.int32)
    tm, le = _sc_pool(idx2d, maskf, llmi, emb_tool, emb_llm)
    return _tc_dense(A_content, le, tm, W_content.T, W_ids.T)


# speedup vs baseline: 8.3627x; 1.2523x over previous
import jax
import jax.numpy as jnp
from jax import lax
from jax.experimental import pallas as pl
from jax.experimental.pallas import tpu as pltpu
from jax.experimental.pallas import tpu_sc as plsc

B = 16384
L = 20
D = 64
DC = 128
TOK = 64

NC = 2
NS = 16
NW = NC * NS
PB = B // NW
CB = 32
NCH = PB // CB
RPC = CB * L
GID = 128
NG = RPC // GID


def _sc_body(idx_hbm, mask_hbm, llmidx_hbm, tool_tab, llm_tab,
             ids_out,
             idx_v, mask_v, rows_v, tm_v, lidx_v, lrows_v, gsems):
    c = lax.axis_index("c")
    s = lax.axis_index("s")
    wid = s * NC + c

    def stage_and_fire(buf, ci):
        base = wid * PB + ci * CB
        pltpu.sync_copy(idx_hbm.at[pl.ds(base * L, RPC)], idx_v.at[buf])
        pltpu.sync_copy(mask_hbm.at[pl.ds(base * L, RPC)],
                        mask_v.at[buf, pl.ds(0, RPC)])
        pltpu.sync_copy(llmidx_hbm.at[pl.ds(base, CB)], lidx_v.at[buf])
        for j in range(NG):
            pltpu.async_copy(
                tool_tab.at[idx_v.at[buf, pl.ds(j * GID, GID)]],
                rows_v.at[buf, pl.ds(j * GID, GID)], gsems.at[buf])
        pltpu.async_copy(llm_tab.at[lidx_v.at[buf]], lrows_v.at[buf],
                         gsems.at[buf])

    def drain(buf):
        for j in range(NG):
            pltpu.make_async_copy(
                tool_tab.at[idx_v.at[buf, pl.ds(j * GID, GID)]],
                rows_v.at[buf, pl.ds(j * GID, GID)], gsems.at[buf]).wait()
        pltpu.make_async_copy(llm_tab.at[lidx_v.at[buf]], lrows_v.at[buf],
                              gsems.at[buf]).wait()

    def compute(buf, ci):
        base = wid * PB + ci * CB

        def bbody(b, bc):
            r0 = b * L
            m0 = mask_v[buf, pl.ds(r0, 16)]
            m1 = mask_v[buf, pl.ds(r0 + 16, 16)]
            ws = [m0[l] for l in range(16)] + [m1[l] for l in range(L - 16)]
            denom = jnp.float32(1e-8)
            for l in range(L):
                denom = denom + ws[l]
            inv = jnp.full((16,), 1.0, jnp.float32) / jnp.broadcast_to(
                denom, (16,))
            accs = [jnp.zeros((16,), jnp.float32) for _ in range(4)]
            for l in range(L):
                for g in range(4):
                    accs[g] = accs[g] + ws[l] * rows_v[buf, r0 + l,
                                                       pl.ds(g * 16, 16)]
            for g in range(4):
                tm_v[buf, b, pl.ds(g * 16, 16)] = accs[g] * inv
            return bc

        lax.fori_loop(0, CB, bbody, 0)
        pltpu.sync_copy(lrows_v.at[buf],
                        ids_out.at[pl.ds(base, CB), pl.ds(0, D)])
        pltpu.sync_copy(tm_v.at[buf],
                        ids_out.at[pl.ds(base, CB), pl.ds(D, D)])

    stage_and_fire(0, 0)

    def pair(g, carry):
        c0 = 2 * g
        c1 = 2 * g + 1
        stage_and_fire(1, c1)
        drain(0)
        compute(0, c0)

        @pl.when(c1 + 1 < NCH)
        def _():
            stage_and_fire(0, c1 + 1)

        drain(1)
        compute(1, c1)
        return carry

    lax.fori_loop(0, NCH // 2, pair, 0)


_sc_pool = pl.kernel(
    _sc_body,
    out_type=jax.ShapeDtypeStruct((B, 2 * D), jnp.float32),
    mesh=plsc.VectorSubcoreMesh(core_axis_name="c", subcore_axis_name="s"),
    compiler_params=pltpu.CompilerParams(use_tc_tiling_on_sc=False),
    scratch_types=[
        pltpu.VMEM((2, RPC), jnp.int32),
        pltpu.VMEM((2, RPC + 16), jnp.float32),
        pltpu.VMEM((2, RPC, D), jnp.float32),
        pltpu.VMEM((2, CB, D), jnp.float32),
        pltpu.VMEM((2, CB), jnp.int32),
        pltpu.VMEM((2, CB, D), jnp.float32),
        pltpu.SemaphoreType.DMA((2,)),
    ],
)


def _tc_body(a_ref, ids_ref, wc_ref, wi_ref, o_ref):
    e = jnp.dot(a_ref[...], wc_ref[...], preferred_element_type=jnp.float32)
    e = e + jnp.dot(ids_ref[...], wi_ref[...],
                    preferred_element_type=jnp.float32)
    n = jnp.sqrt(jnp.sum(e * e, axis=-1, keepdims=True))
    o_ref[...] = e / jnp.maximum(n, 1e-12)


_BLK = 2048


def _tc_dense(A_content, ids, wc_t, wi_t):
    grid = (B // _BLK,)
    return pl.pallas_call(
        _tc_body,
        grid=grid,
        in_specs=[
            pl.BlockSpec((_BLK, DC), lambda i: (i, 0)),
            pl.BlockSpec((_BLK, 2 * D), lambda i: (i, 0)),
            pl.BlockSpec((DC, TOK), lambda i: (0, 0)),
            pl.BlockSpec((2 * D, TOK), lambda i: (0, 0)),
        ],
        out_specs=pl.BlockSpec((_BLK, TOK), lambda i: (i, 0)),
        out_shape=jax.ShapeDtypeStruct((B, TOK), jnp.float32),
    )(A_content, ids, wc_t, wi_t)


def kernel(A_content, tool_idx_pad, tool_mask, llm_idx, emb_tool, emb_llm,
           W_content, W_ids):
    idxf = tool_idx_pad.astype(jnp.int32).reshape(B * L)
    maskf = tool_mask.reshape(B * L)
    llmi = llm_idx.astype(jnp.int32)
    ids = _sc_pool(idxf, maskf, llmi, emb_tool, emb_llm)
    return _tc_dense(A_content, ids, W_content.T, W_ids.T)


# speedup vs baseline: 9.4229x; 1.1268x over previous
import jax
import jax.numpy as jnp
from jax import lax
from jax.experimental import pallas as pl
from jax.experimental.pallas import tpu as pltpu
from jax.experimental.pallas import tpu_sc as plsc

B = 16384
L = 20
D = 64
DC = 128
TOK = 64

NC = 2
NS = 16
NW = NC * NS
PB = B // NW
CB = 32
NCH = PB // CB
RPC = CB * L
GID = 128
NG = RPC // GID


def _sc_body(idx_hbm, mask_hbm, llmidx_hbm, tool_tab, llm_tab,
             ids_out,
             idx_v, mask_v, lidx_v, rows_v, tm_v, lrows_v, gsems):
    c = lax.axis_index("c")
    s = lax.axis_index("s")
    wid = s * NC + c
    wbase = wid * PB

    pltpu.sync_copy(idx_hbm.at[pl.ds(wbase * L, PB * L)], idx_v)
    pltpu.sync_copy(mask_hbm.at[pl.ds(wbase * L, PB * L)],
                    mask_v.at[pl.ds(0, PB * L)])
    pltpu.sync_copy(llmidx_hbm.at[pl.ds(wbase, PB)], lidx_v)

    def fire(buf, ci):
        o = ci * RPC
        for j in range(NG):
            pltpu.async_copy(
                tool_tab.at[idx_v.at[pl.ds(o + j * GID, GID)]],
                rows_v.at[buf, pl.ds(j * GID, GID)], gsems.at[buf])
        pltpu.async_copy(llm_tab.at[lidx_v.at[pl.ds(ci * CB, CB)]],
                         lrows_v.at[buf], gsems.at[buf])

    def drain(buf, ci):
        o = ci * RPC
        for j in range(NG):
            pltpu.make_async_copy(
                tool_tab.at[idx_v.at[pl.ds(o + j * GID, GID)]],
                rows_v.at[buf, pl.ds(j * GID, GID)], gsems.at[buf]).wait()
        pltpu.make_async_copy(llm_tab.at[lidx_v.at[pl.ds(ci * CB, CB)]],
                              lrows_v.at[buf], gsems.at[buf]).wait()

    def compute(buf, ci):
        base = wbase + ci * CB
        mo = ci * RPC

        def bbody(i, bc):
            for b2 in range(2):
                b = i * 2 + b2
                r0 = b * L
                m0 = mask_v[pl.ds(mo + r0, 16)]
                m1 = mask_v[pl.ds(mo + r0 + 16, 16)]
                ws = ([m0[l] for l in range(16)]
                      + [m1[l] for l in range(L - 16)])
                denom = jnp.float32(1e-8)
                for l in range(L):
                    denom = denom + ws[l]
                inv = jnp.full((16,), 1.0, jnp.float32) / jnp.broadcast_to(
                    denom, (16,))
                accs = [jnp.zeros((16,), jnp.float32) for _ in range(4)]
                for l in range(L):
                    for g in range(4):
                        accs[g] = accs[g] + ws[l] * rows_v[buf, r0 + l,
                                                           pl.ds(g * 16, 16)]
                for g in range(4):
                    tm_v[buf, b, pl.ds(g * 16, 16)] = accs[g] * inv
            return bc

        lax.fori_loop(0, CB // 2, bbody, 0)
        pltpu.sync_copy(lrows_v.at[buf],
                        ids_out.at[pl.ds(base, CB), pl.ds(0, D)])
        pltpu.sync_copy(tm_v.at[buf],
                        ids_out.at[pl.ds(base, CB), pl.ds(D, D)])

    fire(0, 0)

    def pair(g, carry):
        c0 = 2 * g
        c1 = 2 * g + 1
        fire(1, c1)
        drain(0, c0)
        compute(0, c0)

        @pl.when(c1 + 1 < NCH)
        def _():
            fire(0, c1 + 1)

        drain(1, c1)
        compute(1, c1)
        return carry

    lax.fori_loop(0, NCH // 2, pair, 0)


_sc_pool = pl.kernel(
    _sc_body,
    out_type=jax.ShapeDtypeStruct((B, 2 * D), jnp.float32),
    mesh=plsc.VectorSubcoreMesh(core_axis_name="c", subcore_axis_name="s"),
    compiler_params=pltpu.CompilerParams(use_tc_tiling_on_sc=False),
    scratch_types=[
        pltpu.VMEM((PB * L,), jnp.int32),
        pltpu.VMEM((PB * L + 16,), jnp.float32),
        pltpu.VMEM((PB,), jnp.int32),
        pltpu.VMEM((2, RPC, D), jnp.float32),
        pltpu.VMEM((2, CB, D), jnp.float32),
        pltpu.VMEM((2, CB, D), jnp.float32),
        pltpu.SemaphoreType.DMA((2,)),
    ],
)


def _tc_body(a_ref, ids_ref, wc_ref, wi_ref, o_ref):
    e = jnp.dot(a_ref[...], wc_ref[...], preferred_element_type=jnp.float32)
    e = e + jnp.dot(ids_ref[...], wi_ref[...],
                    preferred_element_type=jnp.float32)
    n = jnp.sqrt(jnp.sum(e * e, axis=-1, keepdims=True))
    o_ref[...] = e / jnp.maximum(n, 1e-12)


_BLK = 4096


def _tc_dense(A_content, ids, wc_t, wi_t):
    grid = (B // _BLK,)
    return pl.pallas_call(
        _tc_body,
        grid=grid,
        in_specs=[
            pl.BlockSpec((_BLK, DC), lambda i: (i, 0)),
            pl.BlockSpec((_BLK, 2 * D), lambda i: (i, 0)),
            pl.BlockSpec((DC, TOK), lambda i: (0, 0)),
            pl.BlockSpec((2 * D, TOK), lambda i: (0, 0)),
        ],
        out_specs=pl.BlockSpec((_BLK, TOK), lambda i: (i, 0)),
        out_shape=jax.ShapeDtypeStruct((B, TOK), jnp.float32),
    )(A_content, ids, wc_t, wi_t)


def kernel(A_content, tool_idx_pad, tool_mask, llm_idx, emb_tool, emb_llm,
           W_content, W_ids):
    idxf = tool_idx_pad.astype(jnp.int32).reshape(B * L)
    maskf = tool_mask.reshape(B * L)
    llmi = llm_idx.astype(jnp.int32)
    ids = _sc_pool(idxf, maskf, llmi, emb_tool, emb_llm)
    return _tc_dense(A_content, ids, W_content.T, W_ids.T)
